# baseline (speedup 1.0000x reference)
import jax
import jax.numpy as jnp
from jax.experimental import pallas as pl
from jax.experimental.pallas import tpu as pltpu

BATCH = 16384
HIDDEN = 4096
NUM_LAYERS = 32
TOPK = 4
CHUNK = 512
PAIR = 2 * CHUNK


def _gates(logits, hard):
    sig = jax.nn.sigmoid(logits)
    cols = jax.lax.broadcasted_iota(jnp.int32, sig.shape, 1)
    s = sig
    gates = jnp.zeros_like(sig)
    for _ in range(TOPK):
        m = jnp.max(s, axis=1, keepdims=True)
        ismax = s == m
        first = jnp.min(jnp.where(ismax, cols, NUM_LAYERS), axis=1,
                        keepdims=True)
        sel = ismax & (cols == first)
        gates = jnp.where(sel, 1.0, gates)
        s = jnp.where(sel, -jnp.inf, s)
    return jnp.where(hard != 0, gates, sig)


def _router_kernel(hard_ref, ha_ref, hb_ref, w_ref, b_ref, out_ref):
    dn = (((1,), (1,)), ((), ()))
    la = jax.lax.dot_general(ha_ref[0], w_ref[...], dimension_numbers=dn,
                             preferred_element_type=jnp.float32) + b_ref[...]
    lb = jax.lax.dot_general(hb_ref[0], w_ref[...], dimension_numbers=dn,
                             preferred_element_type=jnp.float32) + b_ref[...]
    hard = hard_ref[0]
    out_ref[:CHUNK, :] = _gates(la, hard)
    out_ref[CHUNK:, :] = _gates(lb, hard)


def kernel(h_pooled, W, b, hard):
    hard_arr = jnp.asarray(hard, dtype=jnp.int32).reshape((1,))
    b2 = b.reshape(1, NUM_LAYERS)
    h3 = h_pooled.reshape(BATCH // CHUNK, CHUNK, HIDDEN)
    grid = (BATCH // PAIR,)
    return pl.pallas_call(
        _router_kernel,
        grid_spec=pltpu.PrefetchScalarGridSpec(
            num_scalar_prefetch=1,
            grid=grid,
            in_specs=[
                pl.BlockSpec((1, CHUNK, HIDDEN), lambda i, *_: (2 * i, 0, 0)),
                pl.BlockSpec((1, CHUNK, HIDDEN),
                             lambda i, *_: (2 * i + 1, 0, 0)),
                pl.BlockSpec((NUM_LAYERS, HIDDEN), lambda i, *_: (0, 0)),
                pl.BlockSpec((1, NUM_LAYERS), lambda i, *_: (0, 0)),
            ],
            out_specs=pl.BlockSpec((PAIR, NUM_LAYERS), lambda i, *_: (i, 0)),
        ),
        out_shape=jax.ShapeDtypeStruct((BATCH, NUM_LAYERS), jnp.float32),
    )(hard_arr, h3, h3, W, b2)
